# Initial kernel scaffold; baseline (speedup 1.0000x reference)
import jax
import jax.numpy as jnp
from jax.experimental import pallas as pl


def kernel(query_embedding, memory_bank):
    raise NotImplementedError("write your pallas kernel here")


# speedup vs baseline: 1.6559x; 1.6559x over previous
import functools

import jax
import jax.numpy as jnp
from jax import lax
from jax.experimental import pallas as pl
from jax.experimental.pallas import tpu as pltpu
from jax.experimental.pallas import tpu_sc as plsc

TOPK = 5
QN = 1024
D = 64
N = 100000
MB = 2048
NBLK = -(-N // MB)
NPAD = MB * NBLK
PADK = 8

_NEG = float("-inf")
_BIG = 2**30


def _l2norm(x, eps=1e-12):
    norm = jnp.linalg.norm(x, ord=2, axis=-1, keepdims=True)
    return x / jnp.maximum(norm, eps)


def _topk_body(qn_ref, mt_ref, idx_out_ref, vals_ref, idxs_ref):
    i = pl.program_id(0)

    @pl.when(i == 0)
    def _init():
        vals_ref[...] = jnp.full((QN, PADK), _NEG, jnp.float32)
        idxs_ref[...] = jnp.zeros((QN, PADK), jnp.int32)

    q = qn_ref[...]
    mt = mt_ref[...]
    sims = lax.dot_general(q, mt, (((1,), (0,)), ((), ())),
                           preferred_element_type=jnp.float32)
    base = i * MB
    pos = lax.broadcasted_iota(jnp.int32, (QN, MB), 1)
    sims = jnp.where(pos + base < N, sims, _NEG)

    pos8 = lax.broadcasted_iota(jnp.int32, (QN, PADK), 1)

    bvals = jnp.full((QN, PADK), _NEG, jnp.float32)
    bidxs = jnp.zeros((QN, PADK), jnp.int32)
    s = sims
    for j in range(TOPK):
        mv = jnp.max(s, axis=1, keepdims=True)
        ap = jnp.min(jnp.where(s == mv, pos, _BIG), axis=1, keepdims=True)
        bvals = jnp.where(pos8 == j, mv, bvals)
        bidxs = jnp.where(pos8 == j, ap + base, bidxs)
        s = jnp.where(pos == ap, _NEG, s)

    cvals = jnp.concatenate([vals_ref[...], bvals], axis=1)
    cidxs = jnp.concatenate([idxs_ref[...], bidxs], axis=1)
    pos16 = lax.broadcasted_iota(jnp.int32, (QN, 2 * PADK), 1)
    nvals = jnp.full((QN, PADK), _NEG, jnp.float32)
    nidxs = jnp.zeros((QN, PADK), jnp.int32)
    for j in range(TOPK):
        mv = jnp.max(cvals, axis=1, keepdims=True)
        ap = jnp.min(jnp.where(cvals == mv, pos16, _BIG), axis=1, keepdims=True)
        sel = jnp.sum(jnp.where(pos16 == ap, cidxs, 0), axis=1, keepdims=True)
        nvals = jnp.where(pos8 == j, mv, nvals)
        nidxs = jnp.where(pos8 == j, sel, nidxs)
        cvals = jnp.where(pos16 == ap, _NEG, cvals)
    vals_ref[...] = nvals
    idxs_ref[...] = nidxs

    @pl.when(i == NBLK - 1)
    def _emit():
        idx_out_ref[...] = nidxs


def _topk_indices(qn, mt):
    return pl.pallas_call(
        _topk_body,
        grid=(NBLK,),
        in_specs=[
            pl.BlockSpec((QN, D), lambda i: (0, 0)),
            pl.BlockSpec((D, MB), lambda i: (0, i)),
        ],
        out_specs=pl.BlockSpec((QN, PADK), lambda i: (0, 0)),
        out_shape=jax.ShapeDtypeStruct((QN, PADK), jnp.int32),
        scratch_shapes=[
            pltpu.VMEM((QN, PADK), jnp.float32),
            pltpu.VMEM((QN, PADK), jnp.int32),
        ],
    )(qn, mt)


_NC, _NS = 2, 16
_NW = _NC * _NS
_B = QN * TOPK
_BPW = _B // _NW

@functools.cache
def _sc_gather_fn():
    mesh = plsc.VectorSubcoreMesh(core_axis_name="c", subcore_axis_name="s")

    @functools.partial(
        pl.kernel,
        mesh=mesh,
        out_type=jax.ShapeDtypeStruct((_B, D), jnp.float32),
        scratch_types=[
            pltpu.VMEM((_BPW,), jnp.int32),
            pltpu.VMEM((_BPW, D), jnp.float32),
            pltpu.SemaphoreType.DMA,
        ],
        compiler_params=pltpu.CompilerParams(use_tc_tiling_on_sc=False),
    )
    def _sc_gather(table_hbm, idx_hbm, out_hbm, idx_v, rows_v, sem):
        wid = lax.axis_index("s") * _NC + lax.axis_index("c")
        base = wid * _BPW
        pltpu.sync_copy(idx_hbm.at[pl.ds(base, _BPW)], idx_v)
        pltpu.async_copy(table_hbm.at[idx_v], rows_v, sem).wait()
        pltpu.sync_copy(rows_v, out_hbm.at[pl.ds(base, _BPW)])

    return _sc_gather


def kernel(query_embedding, memory_bank):
    qn = _l2norm(query_embedding)
    mn = _l2norm(memory_bank)
    mt = jnp.pad(mn, ((0, NPAD - N), (0, 0))).T
    idx_pad = _topk_indices(qn, mt)
    idx = idx_pad[:, :TOPK].reshape(_B)
    rows = _sc_gather_fn()(memory_bank, idx)
    return rows.reshape(QN, TOPK, D)


# speedup vs baseline: 2.1216x; 1.2812x over previous
import functools

import jax
import jax.numpy as jnp
from jax import lax
from jax.experimental import pallas as pl
from jax.experimental.pallas import tpu as pltpu
from jax.experimental.pallas import tpu_sc as plsc

TOPK = 5
QN = 1024
D = 64
N = 100000
MB = 2048
NBLK = -(-N // MB)
PADK = 8

_NEG = float("-inf")
_BIGF = 1e30


def _l2norm(x, eps=1e-12):
    norm = jnp.linalg.norm(x, ord=2, axis=-1, keepdims=True)
    return x / jnp.maximum(norm, eps)


def _topk_body(qt_ref, m_ref, idx_out_ref, vals_ref, idxs_ref):
    i = pl.program_id(0)

    @pl.when(i == 0)
    def _init():
        vals_ref[...] = jnp.full((PADK, QN), _NEG, jnp.float32)
        idxs_ref[...] = jnp.zeros((PADK, QN), jnp.float32)

    qt = qt_ref[...]
    m = m_ref[...]
    sims = lax.dot_general(m, qt, (((1,), (0,)), ((), ())),
                           preferred_element_type=jnp.float32)
    basef = (i * MB).astype(jnp.float32)
    pos = lax.broadcasted_iota(jnp.int32, (MB, QN), 0).astype(jnp.float32)
    sims = jnp.where(pos + basef < float(N), sims, _NEG)

    posk = lax.broadcasted_iota(jnp.int32, (PADK, QN), 0).astype(jnp.float32)

    bvals = jnp.full((PADK, QN), _NEG, jnp.float32)
    bidxs = jnp.zeros((PADK, QN), jnp.float32)
    s = sims
    for j in range(TOPK):
        mv = jnp.max(s, axis=0, keepdims=True)
        ap = jnp.min(jnp.where(s == mv, pos, _BIGF), axis=0, keepdims=True)
        bvals = jnp.where(posk == float(j), mv, bvals)
        bidxs = jnp.where(posk == float(j), ap + basef, bidxs)
        if j < TOPK - 1:
            s = jnp.where(pos == ap, _NEG, s)

    cvals = jnp.concatenate([vals_ref[...], bvals], axis=0)
    cidxs = jnp.concatenate([idxs_ref[...], bidxs], axis=0)
    pos2k = lax.broadcasted_iota(jnp.int32, (2 * PADK, QN), 0).astype(jnp.float32)
    nvals = jnp.full((PADK, QN), _NEG, jnp.float32)
    nidxs = jnp.zeros((PADK, QN), jnp.float32)
    for j in range(TOPK):
        mv = jnp.max(cvals, axis=0, keepdims=True)
        ap = jnp.min(jnp.where(cvals == mv, pos2k, _BIGF), axis=0, keepdims=True)
        hit = pos2k == ap
        sel = jnp.sum(jnp.where(hit, cidxs, 0.0), axis=0, keepdims=True)
        nvals = jnp.where(posk == float(j), mv, nvals)
        nidxs = jnp.where(posk == float(j), sel, nidxs)
        if j < TOPK - 1:
            cvals = jnp.where(hit, _NEG, cvals)
    vals_ref[...] = nvals
    idxs_ref[...] = nidxs

    @pl.when(i == NBLK - 1)
    def _emit():
        idx_out_ref[...] = nidxs.astype(jnp.int32)


def _topk_indices(qt, mn):
    return pl.pallas_call(
        _topk_body,
        grid=(NBLK,),
        in_specs=[
            pl.BlockSpec((D, QN), lambda i: (0, 0)),
            pl.BlockSpec((MB, D), lambda i: (i, 0)),
        ],
        out_specs=pl.BlockSpec((PADK, QN), lambda i: (0, 0)),
        out_shape=jax.ShapeDtypeStruct((PADK, QN), jnp.int32),
        scratch_shapes=[
            pltpu.VMEM((PADK, QN), jnp.float32),
            pltpu.VMEM((PADK, QN), jnp.float32),
        ],
    )(qt, mn)


_NC, _NS = 2, 16
_NW = _NC * _NS
_B = QN * TOPK
_BPW = _B // _NW


@functools.cache
def _sc_gather_fn():
    mesh = plsc.VectorSubcoreMesh(core_axis_name="c", subcore_axis_name="s")

    @functools.partial(
        pl.kernel,
        mesh=mesh,
        out_type=jax.ShapeDtypeStruct((_B, D), jnp.float32),
        scratch_types=[
            pltpu.VMEM((_BPW,), jnp.int32),
            pltpu.VMEM((_BPW, D), jnp.float32),
            pltpu.SemaphoreType.DMA,
        ],
        compiler_params=pltpu.CompilerParams(use_tc_tiling_on_sc=False),
    )
    def _sc_gather(table_hbm, idx_hbm, out_hbm, idx_v, rows_v, sem):
        wid = lax.axis_index("s") * _NC + lax.axis_index("c")
        base = wid * _BPW
        pltpu.sync_copy(idx_hbm.at[pl.ds(base, _BPW)], idx_v)
        pltpu.async_copy(table_hbm.at[idx_v], rows_v, sem).wait()
        pltpu.sync_copy(rows_v, out_hbm.at[pl.ds(base, _BPW)])

    return _sc_gather


def kernel(query_embedding, memory_bank):
    qn = _l2norm(query_embedding)
    mn = _l2norm(memory_bank)
    idx_kq = _topk_indices(qn.T, mn)
    idx = idx_kq[:TOPK, :].T.reshape(_B)
    rows = _sc_gather_fn()(memory_bank, idx)
    return rows.reshape(QN, TOPK, D)


# speedup vs baseline: 2.9473x; 1.3892x over previous
import functools

import jax
import jax.numpy as jnp
from jax import lax
from jax.experimental import pallas as pl
from jax.experimental.pallas import tpu as pltpu
from jax.experimental.pallas import tpu_sc as plsc

TOPK = 5
QN = 1024
D = 64
N = 100000
G = 32
MB = 2048
GPB = MB // G
NBLK = -(-N // MB)
NG = NBLK * GPB
T = 8
CPQ = T * G
QT = 128
NQT = QN // QT
CTOT = QN * CPQ
PADK = 8

_NEG = float("-inf")
_BIGF = 1e30


def _l2norm(x, eps=1e-12):
    norm = jnp.linalg.norm(x, ord=2, axis=-1, keepdims=True)
    return x / jnp.maximum(norm, eps)


def _gmax_body(qt_ref, m_ref, gv_ref):
    i = pl.program_id(0)
    qt = qt_ref[...]
    m = m_ref[...]
    sims = lax.dot_general(m, qt, (((1,), (0,)), ((), ())),
                           preferred_element_type=jnp.float32)

    @pl.when(i < NBLK - 1)
    def _full():
        gv_ref[...] = jnp.max(sims.reshape(GPB, G, QN), axis=1)

    @pl.when(i == NBLK - 1)
    def _ragged():
        pos = lax.broadcasted_iota(jnp.int32, (MB, QN), 0)
        s = jnp.where(pos + i * MB < N, sims, _NEG)
        gv_ref[...] = jnp.max(s.reshape(GPB, G, QN), axis=1)


def _gmax(qt, mn):
    return pl.pallas_call(
        _gmax_body,
        grid=(NBLK,),
        in_specs=[
            pl.BlockSpec((D, QN), lambda i: (0, 0)),
            pl.BlockSpec((MB, D), lambda i: (i, 0)),
        ],
        out_specs=pl.BlockSpec((GPB, QN), lambda i: (i, 0)),
        out_shape=jax.ShapeDtypeStruct((NG, QN), jnp.float32),
    )(qt, mn)


def _gsel_body(gv_ref, cidx_ref):
    s = gv_ref[...]
    pos = lax.broadcasted_iota(jnp.int32, (NG, QN), 0).astype(jnp.float32)
    picks = []
    for k in range(T):
        mv = jnp.max(s, axis=0, keepdims=True)
        ap = jnp.min(jnp.where(s == mv, pos, _BIGF), axis=0, keepdims=True)
        picks.append(ap)
        if k < T - 1:
            s = jnp.where(pos == ap, _NEG, s)
    gsel = jnp.concatenate(picks, axis=0)
    rep = jnp.broadcast_to(gsel.reshape(T, 1, QN), (T, G, QN))
    rin = lax.broadcasted_iota(jnp.int32, (T, G, QN), 1).astype(jnp.float32)
    cidx = (rep * float(G) + rin).reshape(CPQ, QN)
    cidx_ref[...] = cidx.astype(jnp.int32)


def _gsel(gv):
    return pl.pallas_call(
        _gsel_body,
        out_shape=jax.ShapeDtypeStruct((CPQ, QN), jnp.int32),
    )(gv)


def _rescore_body(rows_ref, qt_ref, cidx_ref, oidx_ref):
    rows = rows_ref[...]
    qtt = qt_ref[...]
    sims2 = lax.dot_general(rows, qtt, (((1,), (0,)), ((), ())),
                            preferred_element_type=jnp.float32)
    col = lax.broadcasted_iota(jnp.int32, (CPQ, QT), 1)
    v = jnp.full((CPQ, QT), _NEG, jnp.float32)
    for j in range(QT):
        v = jnp.where(col == j, sims2[j * CPQ:(j + 1) * CPQ, :], v)

    gidx = cidx_ref[...].astype(jnp.float32)
    posk = lax.broadcasted_iota(jnp.int32, (PADK, QT), 0)
    nidx = jnp.zeros((PADK, QT), jnp.float32)
    for j in range(TOPK):
        mv = jnp.max(v, axis=0, keepdims=True)
        ap = jnp.min(jnp.where(v == mv, gidx, _BIGF), axis=0, keepdims=True)
        nidx = jnp.where(posk == j, ap, nidx)
        if j < TOPK - 1:
            v = jnp.where(gidx == ap, _NEG, v)
    oidx_ref[...] = nidx.astype(jnp.int32)


def _rescore(rowsg, qt, cidx):
    return pl.pallas_call(
        _rescore_body,
        grid=(NQT,),
        in_specs=[
            pl.BlockSpec((QT * CPQ, D), lambda t: (t, 0)),
            pl.BlockSpec((D, QT), lambda t: (0, t)),
            pl.BlockSpec((CPQ, QT), lambda t: (0, t)),
        ],
        out_specs=pl.BlockSpec((PADK, QT), lambda t: (0, t)),
        out_shape=jax.ShapeDtypeStruct((PADK, QN), jnp.int32),
    )(rowsg, qt, cidx)


_NC, _NS = 2, 16
_NW = _NC * _NS


@functools.cache
def _sc_gather_fn(nrows, chunk):
    bpw = nrows // _NW
    nch = bpw // chunk
    mesh = plsc.VectorSubcoreMesh(core_axis_name="c", subcore_axis_name="s")

    @functools.partial(
        pl.kernel,
        mesh=mesh,
        out_type=jax.ShapeDtypeStruct((nrows, D), jnp.float32),
        scratch_types=[
            pltpu.VMEM((chunk,), jnp.int32),
            pltpu.VMEM((chunk, D), jnp.float32),
            pltpu.SemaphoreType.DMA,
        ],
        compiler_params=pltpu.CompilerParams(use_tc_tiling_on_sc=False),
    )
    def _sc_gather(table_hbm, idx_hbm, out_hbm, idx_v, rows_v, sem):
        wid = lax.axis_index("s") * _NC + lax.axis_index("c")
        base = wid * bpw
        for ch in range(nch):
            o = base + ch * chunk
            pltpu.sync_copy(idx_hbm.at[pl.ds(o, chunk)], idx_v)
            pltpu.async_copy(table_hbm.at[idx_v], rows_v, sem).wait()
            pltpu.sync_copy(rows_v, out_hbm.at[pl.ds(o, chunk)])

    return _sc_gather


def kernel(query_embedding, memory_bank):
    qn = _l2norm(query_embedding)
    mn = _l2norm(memory_bank)
    qt = qn.T
    gv = _gmax(qt, mn)
    cidx = _gsel(gv)
    cflat = cidx.T.reshape(CTOT)
    rowsg = _sc_gather_fn(CTOT, 512)(mn, cflat)
    oidx = _rescore(rowsg, qt, cidx)
    idx = oidx[:TOPK, :].T.reshape(QN * TOPK)
    rows = _sc_gather_fn(QN * TOPK, 160)(memory_bank, idx)
    return rows.reshape(QN, TOPK, D)


# speedup vs baseline: 3.5315x; 1.1982x over previous
import functools

import jax
import jax.numpy as jnp
from jax import lax
from jax.experimental import pallas as pl
from jax.experimental.pallas import tpu as pltpu
from jax.experimental.pallas import tpu_sc as plsc

TOPK = 5
QN = 1024
D = 64
N = 100000
G = 32
MB = 2048
GPB = MB // G
NBLK = -(-N // MB)
NG = NBLK * GPB
T = 5
CPQ = T * G
QT = 128
NQT = QN // QT
CTOT = QN * CPQ
PADK = 8

_NEG = float("-inf")
_BIGF = 1e30


def _l2norm(x, eps=1e-12):
    norm = jnp.linalg.norm(x, ord=2, axis=-1, keepdims=True)
    return x / jnp.maximum(norm, eps)


def _gmax_body(qt_ref, m_ref, gv_ref):
    i = pl.program_id(0)
    qt = qt_ref[...]
    m = m_ref[...]
    sims = lax.dot_general(m, qt, (((1,), (0,)), ((), ())),
                           preferred_element_type=jnp.float32)

    @pl.when(i < NBLK - 1)
    def _full():
        gv_ref[...] = jnp.max(sims.reshape(GPB, G, QN), axis=1)

    @pl.when(i == NBLK - 1)
    def _ragged():
        pos = lax.broadcasted_iota(jnp.int32, (MB, QN), 0)
        s = jnp.where(pos + i * MB < N, sims, _NEG)
        gv_ref[...] = jnp.max(s.reshape(GPB, G, QN), axis=1)


def _gmax(qt, mn):
    return pl.pallas_call(
        _gmax_body,
        grid=(NBLK,),
        in_specs=[
            pl.BlockSpec((D, QN), lambda i: (0, 0)),
            pl.BlockSpec((MB, D), lambda i: (i, 0)),
        ],
        out_specs=pl.BlockSpec((GPB, QN), lambda i: (i, 0)),
        out_shape=jax.ShapeDtypeStruct((NG, QN), jnp.float32),
    )(qt, mn)


def _gsel_body(gv_ref, cidx_ref):
    s = gv_ref[...]
    pos = lax.broadcasted_iota(jnp.int32, (NG, QN), 0).astype(jnp.float32)
    picks = []
    for k in range(T):
        mv = jnp.max(s, axis=0, keepdims=True)
        ap = jnp.min(jnp.where(s == mv, pos, _BIGF), axis=0, keepdims=True)
        picks.append(ap)
        if k < T - 1:
            s = jnp.where(pos == ap, _NEG, s)
    gsel = jnp.concatenate(picks, axis=0)
    rep = jnp.broadcast_to(gsel.reshape(T, 1, QN), (T, G, QN))
    rin = lax.broadcasted_iota(jnp.int32, (T, G, QN), 1).astype(jnp.float32)
    cidx = (rep * float(G) + rin).reshape(CPQ, QN)
    cidx_ref[...] = cidx.astype(jnp.int32)


def _gsel(gv):
    return pl.pallas_call(
        _gsel_body,
        out_shape=jax.ShapeDtypeStruct((CPQ, QN), jnp.int32),
    )(gv)


def _rescore_body(rows_ref, qt_ref, cidx_ref, oidx_ref):
    rows = rows_ref[...]
    qtt = qt_ref[...]
    sims2 = lax.dot_general(rows, qtt, (((1,), (0,)), ((), ())),
                            preferred_element_type=jnp.float32)
    col = lax.broadcasted_iota(jnp.int32, (CPQ, QT), 1)
    v = jnp.full((CPQ, QT), _NEG, jnp.float32)
    for j in range(QT):
        v = jnp.where(col == j, sims2[j * CPQ:(j + 1) * CPQ, :], v)

    gidx = cidx_ref[...].astype(jnp.float32)
    posk = lax.broadcasted_iota(jnp.int32, (PADK, QT), 0)
    nidx = jnp.zeros((PADK, QT), jnp.float32)
    for j in range(TOPK):
        mv = jnp.max(v, axis=0, keepdims=True)
        ap = jnp.min(jnp.where(v == mv, gidx, _BIGF), axis=0, keepdims=True)
        nidx = jnp.where(posk == j, ap, nidx)
        if j < TOPK - 1:
            v = jnp.where(gidx == ap, _NEG, v)
    oidx_ref[...] = nidx.astype(jnp.int32)


def _rescore(rowsg, qt, cidx):
    return pl.pallas_call(
        _rescore_body,
        grid=(NQT,),
        in_specs=[
            pl.BlockSpec((QT * CPQ, D), lambda t: (t, 0)),
            pl.BlockSpec((D, QT), lambda t: (0, t)),
            pl.BlockSpec((CPQ, QT), lambda t: (0, t)),
        ],
        out_specs=pl.BlockSpec((PADK, QT), lambda t: (0, t)),
        out_shape=jax.ShapeDtypeStruct((PADK, QN), jnp.int32),
    )(rowsg, qt, cidx)


_NC, _NS = 2, 16
_NW = _NC * _NS


@functools.cache
def _sc_gather_fn(nrows, chunk):
    bpw = nrows // _NW
    nch = bpw // chunk
    mesh = plsc.VectorSubcoreMesh(core_axis_name="c", subcore_axis_name="s")

    @functools.partial(
        pl.kernel,
        mesh=mesh,
        out_type=jax.ShapeDtypeStruct((nrows, D), jnp.float32),
        scratch_types=[
            pltpu.VMEM((chunk,), jnp.int32),
            pltpu.VMEM((chunk,), jnp.int32),
            pltpu.VMEM((chunk, D), jnp.float32),
            pltpu.VMEM((chunk, D), jnp.float32),
            pltpu.SemaphoreType.DMA,
            pltpu.SemaphoreType.DMA,
        ],
        compiler_params=pltpu.CompilerParams(use_tc_tiling_on_sc=False),
    )
    def _sc_gather(table_hbm, idx_hbm, out_hbm, idx0, idx1, rows0, rows1,
                   sem0, sem1):
        wid = lax.axis_index("s") * _NC + lax.axis_index("c")
        base = wid * bpw
        idxb, rowb, semb = (idx0, idx1), (rows0, rows1), (sem0, sem1)
        pltpu.sync_copy(idx_hbm.at[pl.ds(base, chunk)], idx0)
        handles = [pltpu.async_copy(table_hbm.at[idx0], rows0, sem0)]
        for ch in range(nch):
            if ch + 1 < nch:
                b = (ch + 1) % 2
                pltpu.sync_copy(
                    idx_hbm.at[pl.ds(base + (ch + 1) * chunk, chunk)], idxb[b])
                handles.append(
                    pltpu.async_copy(table_hbm.at[idxb[b]], rowb[b], semb[b]))
            handles[ch].wait()
            pltpu.sync_copy(rowb[ch % 2],
                            out_hbm.at[pl.ds(base + ch * chunk, chunk)])

    return _sc_gather


def kernel(query_embedding, memory_bank):
    qn = _l2norm(query_embedding)
    mn = _l2norm(memory_bank)
    qt = qn.T
    gv = _gmax(qt, mn)
    cidx = _gsel(gv)
    cflat = cidx.T.reshape(CTOT)
    rowsg = _sc_gather_fn(CTOT, 512)(mn, cflat)
    oidx = _rescore(rowsg, qt, cidx)
    idx = oidx[:TOPK, :].T.reshape(QN * TOPK)
    rows = _sc_gather_fn(QN * TOPK, 160)(memory_bank, idx)
    return rows.reshape(QN, TOPK, D)


# speedup vs baseline: 3.5439x; 1.0035x over previous
import functools

import jax
import jax.numpy as jnp
from jax import lax
from jax.experimental import pallas as pl
from jax.experimental.pallas import tpu as pltpu
from jax.experimental.pallas import tpu_sc as plsc

TOPK = 5
QN = 1024
D = 64
N = 100000
G = 32
MB = 4096
GPB = MB // G
NBLK = -(-N // MB)
NG = NBLK * GPB
T = 5
CPQ = T * G
QT = 128
NQT = QN // QT
CTOT = QN * CPQ
PADK = 8

_NEG = float("-inf")
_BIGF = 1e30


def _l2norm(x, eps=1e-12):
    norm = jnp.linalg.norm(x, ord=2, axis=-1, keepdims=True)
    return x / jnp.maximum(norm, eps)


def _gmax_body(qt_ref, m_ref, gv_ref):
    i = pl.program_id(0)
    qt = qt_ref[...]
    m = m_ref[...]
    sims = lax.dot_general(m, qt, (((1,), (0,)), ((), ())),
                           preferred_element_type=jnp.float32)

    @pl.when(i < NBLK - 1)
    def _full():
        gv_ref[...] = jnp.max(sims.reshape(GPB, G, QN), axis=1)

    @pl.when(i == NBLK - 1)
    def _ragged():
        pos = lax.broadcasted_iota(jnp.int32, (MB, QN), 0)
        s = jnp.where(pos + i * MB < N, sims, _NEG)
        gv_ref[...] = jnp.max(s.reshape(GPB, G, QN), axis=1)


def _gmax(qt, mn):
    return pl.pallas_call(
        _gmax_body,
        grid=(NBLK,),
        in_specs=[
            pl.BlockSpec((D, QN), lambda i: (0, 0)),
            pl.BlockSpec((MB, D), lambda i: (i, 0)),
        ],
        out_specs=pl.BlockSpec((GPB, QN), lambda i: (i, 0)),
        out_shape=jax.ShapeDtypeStruct((NG, QN), jnp.float32),
    )(qt, mn)


def _gsel_body(gv_ref, cidx_ref):
    s = gv_ref[...]
    pos = lax.broadcasted_iota(jnp.int32, (NG, QN), 0).astype(jnp.float32)
    picks = []
    for k in range(T):
        mv = jnp.max(s, axis=0, keepdims=True)
        ap = jnp.min(jnp.where(s == mv, pos, _BIGF), axis=0, keepdims=True)
        picks.append(ap)
        if k < T - 1:
            s = jnp.where(pos == ap, _NEG, s)
    gsel = jnp.concatenate(picks, axis=0)
    rep = jnp.broadcast_to(gsel.reshape(T, 1, QN), (T, G, QN))
    rin = lax.broadcasted_iota(jnp.int32, (T, G, QN), 1).astype(jnp.float32)
    cidx = (rep * float(G) + rin).reshape(CPQ, QN)
    cidx_ref[...] = cidx.astype(jnp.int32)


def _gsel(gv):
    return pl.pallas_call(
        _gsel_body,
        out_shape=jax.ShapeDtypeStruct((CPQ, QN), jnp.int32),
    )(gv)


def _rescore_body(rows_ref, qt_ref, cidx_ref, oidx_ref):
    rows = rows_ref[...]
    qtt = qt_ref[...]
    sims2 = lax.dot_general(rows, qtt, (((1,), (0,)), ((), ())),
                            preferred_element_type=jnp.float32)
    col = lax.broadcasted_iota(jnp.int32, (CPQ, QT), 1)
    v = jnp.full((CPQ, QT), _NEG, jnp.float32)
    for j in range(QT):
        v = jnp.where(col == j, sims2[j * CPQ:(j + 1) * CPQ, :], v)

    gidx = cidx_ref[...].astype(jnp.float32)
    posk = lax.broadcasted_iota(jnp.int32, (PADK, QT), 0)
    nidx = jnp.zeros((PADK, QT), jnp.float32)
    for j in range(TOPK):
        mv = jnp.max(v, axis=0, keepdims=True)
        ap = jnp.min(jnp.where(v == mv, gidx, _BIGF), axis=0, keepdims=True)
        nidx = jnp.where(posk == j, ap, nidx)
        if j < TOPK - 1:
            v = jnp.where(gidx == ap, _NEG, v)
    oidx_ref[...] = nidx.astype(jnp.int32)


def _rescore(rowsg, qt, cidx):
    return pl.pallas_call(
        _rescore_body,
        grid=(NQT,),
        in_specs=[
            pl.BlockSpec((QT * CPQ, D), lambda t: (t, 0)),
            pl.BlockSpec((D, QT), lambda t: (0, t)),
            pl.BlockSpec((CPQ, QT), lambda t: (0, t)),
        ],
        out_specs=pl.BlockSpec((PADK, QT), lambda t: (0, t)),
        out_shape=jax.ShapeDtypeStruct((PADK, QN), jnp.int32),
    )(rowsg, qt, cidx)


_NC, _NS = 2, 16
_NW = _NC * _NS


@functools.cache
def _sc_gather_fn(nrows, chunk):
    bpw = nrows // _NW
    nch = bpw // chunk
    mesh = plsc.VectorSubcoreMesh(core_axis_name="c", subcore_axis_name="s")

    @functools.partial(
        pl.kernel,
        mesh=mesh,
        out_type=jax.ShapeDtypeStruct((nrows, D), jnp.float32),
        scratch_types=[
            pltpu.VMEM((chunk,), jnp.int32),
            pltpu.VMEM((chunk,), jnp.int32),
            pltpu.VMEM((chunk, D), jnp.float32),
            pltpu.VMEM((chunk, D), jnp.float32),
            pltpu.SemaphoreType.DMA,
            pltpu.SemaphoreType.DMA,
        ],
        compiler_params=pltpu.CompilerParams(use_tc_tiling_on_sc=False),
    )
    def _sc_gather(table_hbm, idx_hbm, out_hbm, idx0, idx1, rows0, rows1,
                   sem0, sem1):
        wid = lax.axis_index("s") * _NC + lax.axis_index("c")
        base = wid * bpw
        idxb, rowb, semb = (idx0, idx1), (rows0, rows1), (sem0, sem1)
        pltpu.sync_copy(idx_hbm.at[pl.ds(base, chunk)], idx0)
        handles = [pltpu.async_copy(table_hbm.at[idx0], rows0, sem0)]
        for ch in range(nch):
            if ch + 1 < nch:
                b = (ch + 1) % 2
                pltpu.sync_copy(
                    idx_hbm.at[pl.ds(base + (ch + 1) * chunk, chunk)], idxb[b])
                handles.append(
                    pltpu.async_copy(table_hbm.at[idxb[b]], rowb[b], semb[b]))
            handles[ch].wait()
            pltpu.sync_copy(rowb[ch % 2],
                            out_hbm.at[pl.ds(base + ch * chunk, chunk)])

    return _sc_gather


def kernel(query_embedding, memory_bank):
    qn = _l2norm(query_embedding)
    mn = _l2norm(memory_bank)
    qt = qn.T
    gv = _gmax(qt, mn)
    cidx = _gsel(gv)
    cflat = cidx.T.reshape(CTOT)
    rowsg = _sc_gather_fn(CTOT, 512)(mn, cflat)
    oidx = _rescore(rowsg, qt, cidx)
    idx = oidx[:TOPK, :].T.reshape(QN * TOPK)
    rows = _sc_gather_fn(QN * TOPK, 160)(memory_bank, idx)
    return rows.reshape(QN, TOPK, D)
